# baseline (speedup 1.0000x reference)
import functools

import jax
import jax.numpy as jnp
from jax import lax
from jax.experimental import pallas as pl
from jax.experimental.pallas import tpu as pltpu
from jax.experimental.pallas import tpu_sc as plsc

F32 = jnp.float32

_N = 10000
_E = 320000
_EH = _E // 2
_D = 128
_ATOM = 128
_NHALF = _N // 2

_NC = 2
_NS = 16
_NW = _NC * _NS
_NP = 10240
_RPT = _NP // _NS

_GK = 40
_GPW = _EH // _NW
_GCH = _GPW // _GK
_GNBUF = 10
_GQ = 5

_SK = 80
_SPW = _E // _NW
_SCH = _SPW // _SK
_SNBUF = 3
_SQ = 2

_BLK = 2000
_NBLK = _EH // _BLK


@functools.cache
def _sc_kernels():
    mesh = plsc.VectorSubcoreMesh(
        core_axis_name="c", subcore_axis_name="s", num_cores=_NC,
        num_subcores=_NS)

    @functools.partial(
        pl.kernel,
        out_type=jax.ShapeDtypeStruct((_EH, _D), F32),
        mesh=mesh,
        scratch_types=[
            pltpu.VMEM((_GCH, _GK), jnp.int32),
            pltpu.VMEM((_GNBUF, _GK, _D), F32),
            pltpu.SemaphoreType.DMA((_GNBUF,)),
            pltpu.SemaphoreType.DMA((_GNBUF,)),
        ],
    )
    def gather(tab_hbm, idx_hbm, out_hbm, idx_v, bufs, in_sems, out_sems):
        cid = lax.axis_index("c")
        sid = lax.axis_index("s")
        wid = sid * _NC + cid
        pltpu.sync_copy(idx_hbm.at[wid], idx_v)

        def in_desc(ch):
            b = ch % _GNBUF
            return pltpu.make_async_copy(
                tab_hbm.at[idx_v.at[ch]], bufs.at[b], in_sems.at[b])

        def out_desc(ch):
            b = ch % _GNBUF
            base = wid * _GPW + ch * _GK
            return pltpu.make_async_copy(
                bufs.at[b], out_hbm.at[pl.ds(base, _GK)], out_sems.at[b])

        def body(ch, c):
            @pl.when(ch >= _GNBUF)
            def _():
                out_desc(ch - _GNBUF).wait()
            in_desc(ch).start()

            @pl.when(ch >= _GQ)
            def _():
                in_desc(ch - _GQ).wait()
                out_desc(ch - _GQ).start()
            return c

        lax.fori_loop(0, _GCH, body, 0)

        def tail1(i, c):
            ch = _GCH - _GQ + i
            in_desc(ch).wait()
            out_desc(ch).start()
            return c

        lax.fori_loop(0, _GQ, tail1, 0)

        def tail2(i, c):
            out_desc(_GCH - _GNBUF + i).wait()
            return c

        lax.fori_loop(0, _GNBUF, tail2, 0)

    @functools.partial(
        pl.kernel,
        out_type=jax.ShapeDtypeStruct((_NC, _NP, _D), F32),
        mesh=mesh,
        scratch_types=[
            pltpu.VMEM((_SCH, _SK), jnp.int32),
            pltpu.VMEM((_SNBUF, _SK, _D), F32),
            pltpu.VMEM_SHARED((_NP, _D), F32),
            pltpu.SemaphoreType.DMA((_SNBUF,)),
            pltpu.SemaphoreType.DMA((_SNBUF,)),
        ],
    )
    def scatter(msga_hbm, msgb_hbm, dst_hbm, out_hbm, idx_v, bufs, acc,
                in_sems, add_sems):
        cid = lax.axis_index("c")
        sid = lax.axis_index("s")
        wid = sid * _NC + cid
        z16 = jnp.zeros((16,), F32)

        def zrow(i, c):
            for j in range(8):
                bufs[0, i, pl.ds(j * 16, 16)] = z16
            return c

        lax.fori_loop(0, _SK, zrow, 0)

        def zacc(k, c):
            pltpu.sync_copy(bufs.at[0], acc.at[pl.ds(sid * _RPT + k * _SK, _SK)])
            return c

        lax.fori_loop(0, _RPT // _SK, zacc, 0)
        pltpu.sync_copy(dst_hbm.at[wid], idx_v)
        plsc.subcore_barrier()

        def add_start(ch):
            b = ch % _SNBUF
            pltpu.async_copy(
                bufs.at[b], acc.at[idx_v.at[ch]], add_sems.at[b], add=True)

        def add_wait(ch):
            b = ch % _SNBUF
            pltpu.make_async_copy(
                bufs.at[b], acc.at[idx_v.at[ch]], add_sems.at[b]).wait()

        def run_pipeline(msg_ref, base0):
            def in_desc(ch):
                b = ch % _SNBUF
                base = wid * _SPW - base0 + ch * _SK
                return pltpu.make_async_copy(
                    msg_ref.at[pl.ds(base, _SK)], bufs.at[b], in_sems.at[b])

            def body(ch, c):
                @pl.when(ch >= _SNBUF)
                def _():
                    add_wait(ch - _SNBUF)
                in_desc(ch).start()

                @pl.when(ch >= _SQ)
                def _():
                    in_desc(ch - _SQ).wait()
                    add_start(ch - _SQ)
                return c

            lax.fori_loop(0, _SCH, body, 0)

            def tail1(i, c):
                ch = _SCH - _SQ + i
                in_desc(ch).wait()
                add_start(ch)
                return c

            lax.fori_loop(0, _SQ, tail1, 0)

            def tail2(i, c):
                add_wait(_SCH - _SNBUF + i)
                return c

            lax.fori_loop(0, _SNBUF, tail2, 0)

        @pl.when(wid < _NW // 2)
        def _():
            run_pipeline(msga_hbm, 0)

        @pl.when(wid >= _NW // 2)
        def _():
            run_pipeline(msgb_hbm, _EH)

        plsc.subcore_barrier()

        def drain(k, c):
            r = sid * _RPT + k * _SK
            pltpu.sync_copy(acc.at[pl.ds(r, _SK)], out_hbm.at[cid, pl.ds(r, _SK)])
            return c

        lax.fori_loop(0, _RPT // _SK, drain, 0)

    return gather, scatter


def _mm_tc(xx, ww):
    nb = 10

    def kk(x_ref, w_ref, o_ref):
        o_ref[...] = jnp.dot(x_ref[...], w_ref[...],
                             preferred_element_type=F32)

    return pl.pallas_call(
        kk,
        grid=(nb,),
        in_specs=[pl.BlockSpec((_N // nb, _D), lambda i: (i, 0)),
                  pl.BlockSpec((_D, _D), lambda i: (0, 0))],
        out_specs=pl.BlockSpec((_N // nb, _D), lambda i: (i, 0)),
        out_shape=jax.ShapeDtypeStruct((_N, _D), F32),
    )(xx, ww)


def _combine_tc(parts):
    nb = 10

    def kk(p_ref, o_ref):
        o_ref[...] = p_ref[0] + p_ref[1]

    return pl.pallas_call(
        kk,
        grid=(nb,),
        in_specs=[pl.BlockSpec((2, _NP // nb, _D), lambda i: (0, i, 0))],
        out_specs=pl.BlockSpec((_NP // nb, _D), lambda i: (i, 0)),
        out_shape=jax.ShapeDtypeStruct((_NP, _D), F32),
    )(parts)


def _init_tc(g0, e, wib, half):
    off = half * _NBLK

    def kk(g_ref, e_ref, w_ref, inp_ref, m_ref):
        v = g_ref[...] + jnp.dot(e_ref[...], w_ref[...],
                                 preferred_element_type=F32)
        inp_ref[...] = v
        m_ref[...] = jnp.maximum(v, 0.0)

    return pl.pallas_call(
        kk,
        grid=(_NBLK,),
        in_specs=[pl.BlockSpec((_BLK, _D), lambda i: (i, 0)),
                  pl.BlockSpec((_BLK, 16), lambda i: (i + off, 0)),
                  pl.BlockSpec((16, _D), lambda i: (0, 0))],
        out_specs=[pl.BlockSpec((_BLK, _D), lambda i: (i, 0)),
                   pl.BlockSpec((_BLK, _D), lambda i: (i, 0))],
        out_shape=[jax.ShapeDtypeStruct((_EH, _D), F32),
                   jax.ShapeDtypeStruct((_EH, _D), F32)],
    )(g0, e, wib)


def _update_tc(inp, g, m, wh):
    def kk(inp_ref, g_ref, m_ref, w_ref, o_ref):
        mm = m_ref[...]
        up = jnp.roll(mm, -1, axis=0)
        dn = jnp.roll(mm, 1, axis=0)
        ridx = lax.broadcasted_iota(jnp.int32, (_BLK, _D), 0)
        msw = jnp.where(ridx % 2 == 0, up, dn)
        a = jnp.dot(g_ref[...] - msw, w_ref[...], preferred_element_type=F32)
        o_ref[...] = jnp.maximum(inp_ref[...] + a, 0.0)

    return pl.pallas_call(
        kk,
        grid=(_NBLK,),
        in_specs=[pl.BlockSpec((_BLK, _D), lambda i: (i, 0)),
                  pl.BlockSpec((_BLK, _D), lambda i: (i, 0)),
                  pl.BlockSpec((_BLK, _D), lambda i: (i, 0)),
                  pl.BlockSpec((_D, _D), lambda i: (0, 0))],
        out_specs=pl.BlockSpec((_BLK, _D), lambda i: (i, 0)),
        out_shape=jax.ShapeDtypeStruct((_EH, _D), F32),
    )(inp, g, m, wh)


def _final_tc(x, parts, seg8, wo1, wo2, b8, wmlp):
    def kk(x_ref, p_ref, s_ref, wo1_ref, wo2_ref, b_ref, wm_ref, o_ref):
        es = p_ref[0, :_N, :] + p_ref[1, :_N, :]
        h = (jnp.dot(x_ref[...], wo1_ref[...], preferred_element_type=F32)
             + jnp.dot(es, wo2_ref[...], preferred_element_type=F32)
             + b_ref[0:1, :])
        h = jnp.maximum(h, 0.0)
        diff = h[_NHALF:, :] - h[:_NHALF, :]
        t = jnp.maximum(jnp.dot(diff, wm_ref[...],
                                preferred_element_type=F32), 0.0)
        seg = jnp.broadcast_to(s_ref[0:1, :], (128, _NHALF))
        oh = (seg == lax.broadcasted_iota(jnp.int32, (128, _NHALF), 0))
        o_ref[...] = jnp.dot(oh.astype(F32), t, preferred_element_type=F32)

    return pl.pallas_call(
        kk,
        out_shape=jax.ShapeDtypeStruct((128, _D), F32),
    )(x, parts, seg8, wo1, wo2, b8, wmlp)


def kernel(x, e, edge_index, segment_ids, W_i, W_h, W_o, b_o, W_mlp):
    src = edge_index[0]
    dst = edge_index[1]
    srca = src[:_EH].reshape(_NW, _GCH, _GK)
    srcb = src[_EH:].reshape(_NW, _GCH, _GK)
    dst3 = dst.reshape(_NW, _SCH, _SK)
    seg8 = jnp.tile(segment_ids[None, :], (8, 1))
    b8 = jnp.tile(b_o[None, :], (8, 1))

    gather, scatter = _sc_kernels()

    a = _mm_tc(x, W_i[:_ATOM])
    g0a = gather(a, srca)
    g0b = gather(a, srcb)
    inpa, ma = _init_tc(g0a, e, W_i[_ATOM:], 0)
    inpb, mb = _init_tc(g0b, e, W_i[_ATOM:], 1)
    for _ in range(4):
        parts = scatter(ma, mb, dst3)
        es = _combine_tc(parts)
        ga = gather(es, srca)
        gb = gather(es, srcb)
        ma = _update_tc(inpa, ga, ma, W_h)
        mb = _update_tc(inpb, gb, mb, W_h)
    parts = scatter(ma, mb, dst3)
    out = _final_tc(x, parts, seg8, W_o[:_ATOM], W_o[_ATOM:], b8, W_mlp)
    return out[:100]


# speedup vs baseline: 1.0628x; 1.0628x over previous
import functools

import jax
import jax.numpy as jnp
from jax import lax
from jax.experimental import pallas as pl
from jax.experimental.pallas import tpu as pltpu
from jax.experimental.pallas import tpu_sc as plsc

F32 = jnp.float32
_CP = pltpu.CompilerParams(skip_device_barrier=True)

_N = 10000
_E = 320000
_EH = _E // 2
_D = 128
_ATOM = 128
_NHALF = _N // 2

_NC = 2
_NS = 16
_NW = _NC * _NS
_NP = 10240
_RPT = _NP // _NS

_GK = 40
_GPW = _EH // _NW
_GCH = _GPW // _GK
_GNBUF = 10
_GQ = 5

_SK = 80
_SPW = _E // _NW
_SCH = _SPW // _SK
_SNBUF = 3
_SQ = 2

_BLK = 2000
_NBLK = _EH // _BLK


@functools.cache
def _sc_kernels():
    mesh = plsc.VectorSubcoreMesh(
        core_axis_name="c", subcore_axis_name="s", num_cores=_NC,
        num_subcores=_NS)

    @functools.partial(
        pl.kernel,
        out_type=jax.ShapeDtypeStruct((_EH, _D), F32),
        mesh=mesh,
        compiler_params=_CP,
        scratch_types=[
            pltpu.VMEM((_GCH, _GK), jnp.int32),
            pltpu.VMEM((_GNBUF, _GK, _D), F32),
            pltpu.SemaphoreType.DMA((_GNBUF,)),
            pltpu.SemaphoreType.DMA((_GNBUF,)),
        ],
    )
    def gather(tab_hbm, idx_hbm, out_hbm, idx_v, bufs, in_sems, out_sems):
        cid = lax.axis_index("c")
        sid = lax.axis_index("s")
        wid = sid * _NC + cid
        pltpu.sync_copy(idx_hbm.at[wid], idx_v)

        def in_desc(ch):
            b = ch % _GNBUF
            return pltpu.make_async_copy(
                tab_hbm.at[idx_v.at[ch]], bufs.at[b], in_sems.at[b])

        def out_desc(ch):
            b = ch % _GNBUF
            base = wid * _GPW + ch * _GK
            return pltpu.make_async_copy(
                bufs.at[b], out_hbm.at[pl.ds(base, _GK)], out_sems.at[b])

        def body(ch, c):
            @pl.when(ch >= _GNBUF)
            def _():
                out_desc(ch - _GNBUF).wait()
            in_desc(ch).start()

            @pl.when(ch >= _GQ)
            def _():
                in_desc(ch - _GQ).wait()
                out_desc(ch - _GQ).start()
            return c

        lax.fori_loop(0, _GCH, body, 0)

        def tail1(i, c):
            ch = _GCH - _GQ + i
            in_desc(ch).wait()
            out_desc(ch).start()
            return c

        lax.fori_loop(0, _GQ, tail1, 0)

        def tail2(i, c):
            out_desc(_GCH - _GNBUF + i).wait()
            return c

        lax.fori_loop(0, _GNBUF, tail2, 0)

    @functools.partial(
        pl.kernel,
        out_type=jax.ShapeDtypeStruct((_NC, _NP, _D), F32),
        mesh=mesh,
        compiler_params=_CP,
        scratch_types=[
            pltpu.VMEM((_SCH, _SK), jnp.int32),
            pltpu.VMEM((_SNBUF, _SK, _D), F32),
            pltpu.VMEM_SHARED((_NP, _D), F32),
            pltpu.SemaphoreType.DMA((_SNBUF,)),
            pltpu.SemaphoreType.DMA((_SNBUF,)),
        ],
    )
    def scatter(msga_hbm, msgb_hbm, dst_hbm, out_hbm, idx_v, bufs, acc,
                in_sems, add_sems):
        cid = lax.axis_index("c")
        sid = lax.axis_index("s")
        wid = sid * _NC + cid
        z16 = jnp.zeros((16,), F32)

        def zrow(i, c):
            for j in range(8):
                bufs[0, i, pl.ds(j * 16, 16)] = z16
            return c

        lax.fori_loop(0, _SK, zrow, 0)

        def zacc(k, c):
            pltpu.sync_copy(bufs.at[0], acc.at[pl.ds(sid * _RPT + k * _SK, _SK)])
            return c

        lax.fori_loop(0, _RPT // _SK, zacc, 0)
        pltpu.sync_copy(dst_hbm.at[wid], idx_v)
        plsc.subcore_barrier()

        def add_start(ch):
            b = ch % _SNBUF
            pltpu.async_copy(
                bufs.at[b], acc.at[idx_v.at[ch]], add_sems.at[b], add=True)

        def add_wait(ch):
            b = ch % _SNBUF
            pltpu.make_async_copy(
                bufs.at[b], acc.at[idx_v.at[ch]], add_sems.at[b]).wait()

        def run_pipeline(msg_ref, base0):
            def in_desc(ch):
                b = ch % _SNBUF
                base = wid * _SPW - base0 + ch * _SK
                return pltpu.make_async_copy(
                    msg_ref.at[pl.ds(base, _SK)], bufs.at[b], in_sems.at[b])

            def body(ch, c):
                @pl.when(ch >= _SNBUF)
                def _():
                    add_wait(ch - _SNBUF)
                in_desc(ch).start()

                @pl.when(ch >= _SQ)
                def _():
                    in_desc(ch - _SQ).wait()
                    add_start(ch - _SQ)
                return c

            lax.fori_loop(0, _SCH, body, 0)

            def tail1(i, c):
                ch = _SCH - _SQ + i
                in_desc(ch).wait()
                add_start(ch)
                return c

            lax.fori_loop(0, _SQ, tail1, 0)

            def tail2(i, c):
                add_wait(_SCH - _SNBUF + i)
                return c

            lax.fori_loop(0, _SNBUF, tail2, 0)

        @pl.when(wid < _NW // 2)
        def _():
            run_pipeline(msga_hbm, 0)

        @pl.when(wid >= _NW // 2)
        def _():
            run_pipeline(msgb_hbm, _EH)

        plsc.subcore_barrier()

        def drain(k, c):
            r = sid * _RPT + k * _SK
            pltpu.sync_copy(acc.at[pl.ds(r, _SK)], out_hbm.at[cid, pl.ds(r, _SK)])
            return c

        lax.fori_loop(0, _RPT // _SK, drain, 0)

    return gather, scatter


def _mm_tc(xx, ww):
    nb = 10

    def kk(x_ref, w_ref, o_ref):
        o_ref[...] = jnp.dot(x_ref[...], w_ref[...],
                             preferred_element_type=F32)

    return pl.pallas_call(
        kk,
        compiler_params=_CP,
        grid=(nb,),
        in_specs=[pl.BlockSpec((_N // nb, _D), lambda i: (i, 0)),
                  pl.BlockSpec((_D, _D), lambda i: (0, 0))],
        out_specs=pl.BlockSpec((_N // nb, _D), lambda i: (i, 0)),
        out_shape=jax.ShapeDtypeStruct((_N, _D), F32),
    )(xx, ww)


def _combine_tc(parts):
    nb = 10

    def kk(p_ref, o_ref):
        o_ref[...] = p_ref[0] + p_ref[1]

    return pl.pallas_call(
        kk,
        compiler_params=_CP,
        grid=(nb,),
        in_specs=[pl.BlockSpec((2, _NP // nb, _D), lambda i: (0, i, 0))],
        out_specs=pl.BlockSpec((_NP // nb, _D), lambda i: (i, 0)),
        out_shape=jax.ShapeDtypeStruct((_NP, _D), F32),
    )(parts)


def _init_tc(g0, e, wib, half):
    off = half * _NBLK

    def kk(g_ref, e_ref, w_ref, inp_ref, m_ref):
        v = g_ref[...] + jnp.dot(e_ref[...], w_ref[...],
                                 preferred_element_type=F32)
        inp_ref[...] = v
        m_ref[...] = jnp.maximum(v, 0.0)

    return pl.pallas_call(
        kk,
        compiler_params=_CP,
        grid=(_NBLK,),
        in_specs=[pl.BlockSpec((_BLK, _D), lambda i: (i, 0)),
                  pl.BlockSpec((_BLK, 16), lambda i: (i + off, 0)),
                  pl.BlockSpec((16, _D), lambda i: (0, 0))],
        out_specs=[pl.BlockSpec((_BLK, _D), lambda i: (i, 0)),
                   pl.BlockSpec((_BLK, _D), lambda i: (i, 0))],
        out_shape=[jax.ShapeDtypeStruct((_EH, _D), F32),
                   jax.ShapeDtypeStruct((_EH, _D), F32)],
    )(g0, e, wib)


def _update_tc(inp, g, m, wh):
    def kk(inp_ref, g_ref, m_ref, w_ref, o_ref):
        mm = m_ref[...]
        up = jnp.roll(mm, -1, axis=0)
        dn = jnp.roll(mm, 1, axis=0)
        ridx = lax.broadcasted_iota(jnp.int32, (_BLK, _D), 0)
        msw = jnp.where(ridx % 2 == 0, up, dn)
        a = jnp.dot(g_ref[...] - msw, w_ref[...], preferred_element_type=F32)
        o_ref[...] = jnp.maximum(inp_ref[...] + a, 0.0)

    return pl.pallas_call(
        kk,
        compiler_params=_CP,
        grid=(_NBLK,),
        in_specs=[pl.BlockSpec((_BLK, _D), lambda i: (i, 0)),
                  pl.BlockSpec((_BLK, _D), lambda i: (i, 0)),
                  pl.BlockSpec((_BLK, _D), lambda i: (i, 0)),
                  pl.BlockSpec((_D, _D), lambda i: (0, 0))],
        out_specs=pl.BlockSpec((_BLK, _D), lambda i: (i, 0)),
        out_shape=jax.ShapeDtypeStruct((_EH, _D), F32),
    )(inp, g, m, wh)


def _final_tc(x, parts, seg8, wo1, wo2, b8, wmlp):
    def kk(x_ref, p_ref, s_ref, wo1_ref, wo2_ref, b_ref, wm_ref, o_ref):
        es = p_ref[0, :_N, :] + p_ref[1, :_N, :]
        h = (jnp.dot(x_ref[...], wo1_ref[...], preferred_element_type=F32)
             + jnp.dot(es, wo2_ref[...], preferred_element_type=F32)
             + b_ref[0:1, :])
        h = jnp.maximum(h, 0.0)
        diff = h[_NHALF:, :] - h[:_NHALF, :]
        t = jnp.maximum(jnp.dot(diff, wm_ref[...],
                                preferred_element_type=F32), 0.0)
        seg = jnp.broadcast_to(s_ref[0:1, :], (128, _NHALF))
        oh = (seg == lax.broadcasted_iota(jnp.int32, (128, _NHALF), 0))
        o_ref[...] = jnp.dot(oh.astype(F32), t, preferred_element_type=F32)

    return pl.pallas_call(
        kk,
        compiler_params=_CP,
        out_shape=jax.ShapeDtypeStruct((128, _D), F32),
    )(x, parts, seg8, wo1, wo2, b8, wmlp)


def kernel(x, e, edge_index, segment_ids, W_i, W_h, W_o, b_o, W_mlp):
    src = edge_index[0]
    dst = edge_index[1]
    srca = src[:_EH].reshape(_NW, _GCH, _GK)
    srcb = src[_EH:].reshape(_NW, _GCH, _GK)
    dst3 = dst.reshape(_NW, _SCH, _SK)
    seg8 = jnp.tile(segment_ids[None, :], (8, 1))
    b8 = jnp.tile(b_o[None, :], (8, 1))

    gather, scatter = _sc_kernels()

    a = _mm_tc(x, W_i[:_ATOM])
    g0a = gather(a, srca)
    g0b = gather(a, srcb)
    inpa, ma = _init_tc(g0a, e, W_i[_ATOM:], 0)
    inpb, mb = _init_tc(g0b, e, W_i[_ATOM:], 1)
    for _ in range(4):
        parts = scatter(ma, mb, dst3)
        es = _combine_tc(parts)
        ga = gather(es, srca)
        gb = gather(es, srcb)
        ma = _update_tc(inpa, ga, ma, W_h)
        mb = _update_tc(inpb, gb, mb, W_h)
    parts = scatter(ma, mb, dst3)
    out = _final_tc(x, parts, seg8, W_o[:_ATOM], W_o[_ATOM:], b8, W_mlp)
    return out[:100]


# speedup vs baseline: 1.0851x; 1.0210x over previous
import functools

import jax
import jax.numpy as jnp
from jax import lax
from jax.experimental import pallas as pl
from jax.experimental.pallas import tpu as pltpu
from jax.experimental.pallas import tpu_sc as plsc

F32 = jnp.float32
_CP = pltpu.CompilerParams(skip_device_barrier=True)

_N = 10000
_E = 320000
_D = 128
_ATOM = 128
_NHALF = _N // 2

_NC = 2
_NS = 16
_NW = _NC * _NS
_NP = 10240
_RPT = _NP // _NS

_K = 80
_PER_W = _E // _NW
_CH = _PER_W // _K

_GNBUF = 8
_GQ = 4
_SNBUF = 4
_SQ = 2

_BLK = 2560
_NBLK = _E // _BLK


@functools.cache
def _sc_kernels():
    mesh = plsc.VectorSubcoreMesh(
        core_axis_name="c", subcore_axis_name="s", num_cores=_NC,
        num_subcores=_NS)

    @functools.partial(
        pl.kernel,
        out_type=jax.ShapeDtypeStruct((_E, _D), F32),
        mesh=mesh,
        compiler_params=_CP,
        scratch_types=[
            pltpu.VMEM((_CH, _K), jnp.int32),
            pltpu.VMEM((_GNBUF, _K, _D), F32),
            pltpu.SemaphoreType.DMA((_GNBUF,)),
            pltpu.SemaphoreType.DMA((_GNBUF,)),
        ],
    )
    def gather(tab_hbm, idx_hbm, out_hbm, idx_v, bufs, in_sems, out_sems):
        cid = lax.axis_index("c")
        sid = lax.axis_index("s")
        wid = sid * _NC + cid
        pltpu.sync_copy(idx_hbm.at[wid], idx_v)

        def in_desc(ch):
            b = ch % _GNBUF
            return pltpu.make_async_copy(
                tab_hbm.at[idx_v.at[ch]], bufs.at[b], in_sems.at[b])

        def out_desc(ch):
            b = ch % _GNBUF
            base = wid * _PER_W + ch * _K
            return pltpu.make_async_copy(
                bufs.at[b], out_hbm.at[pl.ds(base, _K)], out_sems.at[b])

        def body(ch, c):
            @pl.when(ch >= _GNBUF)
            def _():
                out_desc(ch - _GNBUF).wait()
            in_desc(ch).start()

            @pl.when(ch >= _GQ)
            def _():
                in_desc(ch - _GQ).wait()
                out_desc(ch - _GQ).start()
            return c

        lax.fori_loop(0, _CH, body, 0)

        def tail1(i, c):
            ch = _CH - _GQ + i
            in_desc(ch).wait()
            out_desc(ch).start()
            return c

        lax.fori_loop(0, _GQ, tail1, 0)

        def tail2(i, c):
            out_desc(_CH - _GNBUF + i).wait()
            return c

        lax.fori_loop(0, _GNBUF, tail2, 0)

    @functools.partial(
        pl.kernel,
        out_type=jax.ShapeDtypeStruct((_NC, _NP, _D), F32),
        mesh=mesh,
        compiler_params=_CP,
        scratch_types=[
            pltpu.VMEM((_SNBUF, 1, _K), jnp.int32),
            pltpu.VMEM((_SNBUF, _K, _D), F32),
            pltpu.VMEM_SHARED((_NP, _D), F32),
            pltpu.SemaphoreType.DMA((_SNBUF,)),
            pltpu.SemaphoreType.DMA((_SNBUF,)),
            pltpu.SemaphoreType.DMA((_SNBUF,)),
        ],
    )
    def scatter(msg_hbm, dst_hbm, out_hbm, idx_v, bufs, acc,
                idx_sems, in_sems, add_sems):
        cid = lax.axis_index("c")
        sid = lax.axis_index("s")
        wid = sid * _NC + cid
        z16 = jnp.zeros((16,), F32)

        def zrow(i, c):
            for j in range(8):
                bufs[0, i, pl.ds(j * 16, 16)] = z16
            return c

        lax.fori_loop(0, _K, zrow, 0)

        def zacc(k, c):
            pltpu.sync_copy(bufs.at[0], acc.at[pl.ds(sid * _RPT + k * _K, _K)])
            return c

        lax.fori_loop(0, _RPT // _K, zacc, 0)
        plsc.subcore_barrier()

        def idx_desc(ch):
            b = ch % _SNBUF
            return pltpu.make_async_copy(
                dst_hbm.at[wid, pl.ds(ch, 1)], idx_v.at[b], idx_sems.at[b])

        def in_desc(ch):
            b = ch % _SNBUF
            base = wid * _PER_W + ch * _K
            return pltpu.make_async_copy(
                msg_hbm.at[pl.ds(base, _K)], bufs.at[b], in_sems.at[b])

        def add_start(ch):
            b = ch % _SNBUF
            pltpu.async_copy(
                bufs.at[b], acc.at[idx_v.at[b, 0]], add_sems.at[b], add=True)

        def add_wait(ch):
            b = ch % _SNBUF
            pltpu.make_async_copy(
                bufs.at[b], acc.at[idx_v.at[b, 0]], add_sems.at[b]).wait()

        def body(ch, c):
            @pl.when(ch >= _SNBUF)
            def _():
                add_wait(ch - _SNBUF)
            idx_desc(ch).start()
            in_desc(ch).start()

            @pl.when(ch >= _SQ)
            def _():
                idx_desc(ch - _SQ).wait()
                in_desc(ch - _SQ).wait()
                add_start(ch - _SQ)
            return c

        lax.fori_loop(0, _CH, body, 0)

        def tail1(i, c):
            ch = _CH - _SQ + i
            idx_desc(ch).wait()
            in_desc(ch).wait()
            add_start(ch)
            return c

        lax.fori_loop(0, _SQ, tail1, 0)

        def tail2(i, c):
            add_wait(_CH - _SNBUF + i)
            return c

        lax.fori_loop(0, _SNBUF, tail2, 0)
        plsc.subcore_barrier()

        def drain(k, c):
            r = sid * _RPT + k * _K
            pltpu.sync_copy(acc.at[pl.ds(r, _K)], out_hbm.at[cid, pl.ds(r, _K)])
            return c

        lax.fori_loop(0, _RPT // _K, drain, 0)

    return gather, scatter


def _mm_tc(xx, ww):
    nb = 10

    def kk(x_ref, w_ref, o_ref):
        o_ref[...] = jnp.dot(x_ref[...], w_ref[...],
                             preferred_element_type=F32)

    return pl.pallas_call(
        kk,
        compiler_params=_CP,
        grid=(nb,),
        in_specs=[pl.BlockSpec((_N // nb, _D), lambda i: (i, 0)),
                  pl.BlockSpec((_D, _D), lambda i: (0, 0))],
        out_specs=pl.BlockSpec((_N // nb, _D), lambda i: (i, 0)),
        out_shape=jax.ShapeDtypeStruct((_N, _D), F32),
    )(xx, ww)


def _combine_tc(parts):
    nb = 10

    def kk(p_ref, o_ref):
        o_ref[...] = p_ref[0] + p_ref[1]

    return pl.pallas_call(
        kk,
        compiler_params=_CP,
        grid=(nb,),
        in_specs=[pl.BlockSpec((2, _NP // nb, _D), lambda i: (0, i, 0))],
        out_specs=pl.BlockSpec((_NP // nb, _D), lambda i: (i, 0)),
        out_shape=jax.ShapeDtypeStruct((_NP, _D), F32),
    )(parts)


def _init_tc(g0, e, wib):
    def kk(g_ref, e_ref, w_ref, inp_ref, m_ref):
        v = g_ref[...] + jnp.dot(e_ref[...], w_ref[...],
                                 preferred_element_type=F32)
        inp_ref[...] = v
        m_ref[...] = jnp.maximum(v, 0.0)

    return pl.pallas_call(
        kk,
        compiler_params=_CP,
        grid=(_NBLK,),
        in_specs=[pl.BlockSpec((_BLK, _D), lambda i: (i, 0)),
                  pl.BlockSpec((_BLK, 16), lambda i: (i, 0)),
                  pl.BlockSpec((16, _D), lambda i: (0, 0))],
        out_specs=[pl.BlockSpec((_BLK, _D), lambda i: (i, 0)),
                   pl.BlockSpec((_BLK, _D), lambda i: (i, 0))],
        out_shape=[jax.ShapeDtypeStruct((_E, _D), F32),
                   jax.ShapeDtypeStruct((_E, _D), F32)],
    )(g0, e, wib)


def _update_tc(inp, g, m, wh):
    def kk(inp_ref, g_ref, m_ref, w_ref, o_ref):
        mm = m_ref[...]
        up = jnp.roll(mm, -1, axis=0)
        dn = jnp.roll(mm, 1, axis=0)
        ridx = lax.broadcasted_iota(jnp.int32, (_BLK, _D), 0)
        msw = jnp.where(ridx % 2 == 0, up, dn)
        a = jnp.dot(g_ref[...] - msw, w_ref[...], preferred_element_type=F32)
        o_ref[...] = jnp.maximum(inp_ref[...] + a, 0.0)

    return pl.pallas_call(
        kk,
        compiler_params=_CP,
        grid=(_NBLK,),
        in_specs=[pl.BlockSpec((_BLK, _D), lambda i: (i, 0)),
                  pl.BlockSpec((_BLK, _D), lambda i: (i, 0)),
                  pl.BlockSpec((_BLK, _D), lambda i: (i, 0)),
                  pl.BlockSpec((_D, _D), lambda i: (0, 0))],
        out_specs=pl.BlockSpec((_BLK, _D), lambda i: (i, 0)),
        out_shape=jax.ShapeDtypeStruct((_E, _D), F32),
    )(inp, g, m, wh)


def _final_tc(x, parts, seg8, wo1, wo2, b8, wmlp):
    def kk(x_ref, p_ref, s_ref, wo1_ref, wo2_ref, b_ref, wm_ref, o_ref):
        es = p_ref[0, :_N, :] + p_ref[1, :_N, :]
        h = (jnp.dot(x_ref[...], wo1_ref[...], preferred_element_type=F32)
             + jnp.dot(es, wo2_ref[...], preferred_element_type=F32)
             + b_ref[0:1, :])
        h = jnp.maximum(h, 0.0)
        diff = h[_NHALF:, :] - h[:_NHALF, :]
        t = jnp.maximum(jnp.dot(diff, wm_ref[...],
                                preferred_element_type=F32), 0.0)
        seg = jnp.broadcast_to(s_ref[0:1, :], (128, _NHALF))
        oh = (seg == lax.broadcasted_iota(jnp.int32, (128, _NHALF), 0))
        o_ref[...] = jnp.dot(oh.astype(F32), t, preferred_element_type=F32)

    return pl.pallas_call(
        kk,
        compiler_params=_CP,
        out_shape=jax.ShapeDtypeStruct((128, _D), F32),
    )(x, parts, seg8, wo1, wo2, b8, wmlp)


def kernel(x, e, edge_index, segment_ids, W_i, W_h, W_o, b_o, W_mlp):
    src = edge_index[0]
    dst = edge_index[1]
    src3 = src.reshape(_NW, _CH, _K)
    dst3 = dst.reshape(_NW, _CH, _K)
    seg8 = jnp.tile(segment_ids[None, :], (8, 1))
    b8 = jnp.tile(b_o[None, :], (8, 1))

    gather, scatter = _sc_kernels()

    a = _mm_tc(x, W_i[:_ATOM])
    g0 = gather(a, src3)
    inp, m = _init_tc(g0, e, W_i[_ATOM:])
    for _ in range(4):
        parts = scatter(m, dst3)
        es = _combine_tc(parts)
        g = gather(es, src3)
        m = _update_tc(inp, g, m, W_h)
    parts = scatter(m, dst3)
    out = _final_tc(x, parts, seg8, W_o[:_ATOM], W_o[_ATOM:], b8, W_mlp)
    return out[:100]


# speedup vs baseline: 1.1147x; 1.0273x over previous
import functools

import jax
import jax.numpy as jnp
from jax import lax
from jax.experimental import pallas as pl
from jax.experimental.pallas import tpu as pltpu
from jax.experimental.pallas import tpu_sc as plsc

F32 = jnp.float32
_CP = pltpu.CompilerParams(skip_device_barrier=True)

_N = 10000
_E = 320000
_D = 128
_ATOM = 128
_NHALF = _N // 2

_NC = 2
_NS = 16
_NW = _NC * _NS
_NP = 10240
_RPT = _NP // _NS

_K = 80
_PER_W = _E // _NW
_CH = _PER_W // _K

_GNBUF = 8
_GQ = 4
_SNBUF = 3
_SQ = 2

_BLK = 3200
_NBLK = _E // _BLK


@functools.cache
def _sc_kernels():
    mesh = plsc.VectorSubcoreMesh(
        core_axis_name="c", subcore_axis_name="s", num_cores=_NC,
        num_subcores=_NS)

    def make_gather(ncols, dtype, nbuf, q):
        @functools.partial(
            pl.kernel,
            out_type=jax.ShapeDtypeStruct((_E, ncols), dtype),
            mesh=mesh,
            compiler_params=_CP,
            scratch_types=[
                pltpu.VMEM((_CH, _K), jnp.int32),
                pltpu.VMEM((nbuf, _K, ncols), dtype),
                pltpu.SemaphoreType.DMA((nbuf,)),
                pltpu.SemaphoreType.DMA((nbuf,)),
            ],
        )
        def gather(tab_hbm, idx_hbm, out_hbm, idx_v, bufs, in_sems, out_sems):
            cid = lax.axis_index("c")
            sid = lax.axis_index("s")
            wid = sid * _NC + cid
            pltpu.sync_copy(idx_hbm.at[wid], idx_v)

            def in_desc(ch):
                b = ch % nbuf
                return pltpu.make_async_copy(
                    tab_hbm.at[idx_v.at[ch]], bufs.at[b], in_sems.at[b])

            def out_desc(ch):
                b = ch % nbuf
                base = wid * _PER_W + ch * _K
                return pltpu.make_async_copy(
                    bufs.at[b], out_hbm.at[pl.ds(base, _K)], out_sems.at[b])

            def body(ch, c):
                @pl.when(ch >= nbuf)
                def _():
                    out_desc(ch - nbuf).wait()
                in_desc(ch).start()

                @pl.when(ch >= q)
                def _():
                    in_desc(ch - q).wait()
                    out_desc(ch - q).start()
                return c

            lax.fori_loop(0, _CH, body, 0)

            def tail1(i, c):
                ch = _CH - q + i
                in_desc(ch).wait()
                out_desc(ch).start()
                return c

            lax.fori_loop(0, q, tail1, 0)

            def tail2(i, c):
                out_desc(_CH - nbuf + i).wait()
                return c

            lax.fori_loop(0, nbuf, tail2, 0)

        return gather

    gather_f32 = make_gather(_D, F32, _GNBUF, _GQ)

    @functools.partial(
        pl.kernel,
        out_type=jax.ShapeDtypeStruct((_NC, _NP, _D), F32),
        mesh=mesh,
        compiler_params=_CP,
        scratch_types=[
            pltpu.VMEM((_CH, _K), jnp.int32),
            pltpu.VMEM((_SNBUF, _K, _D), F32),
            pltpu.VMEM_SHARED((_NP, _D), F32),
            pltpu.SemaphoreType.DMA((_SNBUF,)),
            pltpu.SemaphoreType.DMA((_SNBUF,)),
        ],
    )
    def scatter(msg_hbm, dst_hbm, out_hbm, idx_v, bufs, acc,
                in_sems, add_sems):
        cid = lax.axis_index("c")
        sid = lax.axis_index("s")
        wid = sid * _NC + cid
        z16 = jnp.zeros((16,), F32)

        def zrow(i, c):
            for j in range(8):
                bufs[0, i, pl.ds(j * 16, 16)] = z16
            return c

        lax.fori_loop(0, _K, zrow, 0)

        def zacc(k, c):
            pltpu.sync_copy(bufs.at[0], acc.at[pl.ds(sid * _RPT + k * _K, _K)])
            return c

        lax.fori_loop(0, _RPT // _K, zacc, 0)
        pltpu.sync_copy(dst_hbm.at[wid], idx_v)
        plsc.subcore_barrier()

        def in_desc(ch):
            b = ch % _SNBUF
            base = wid * _PER_W + ch * _K
            return pltpu.make_async_copy(
                msg_hbm.at[pl.ds(base, _K)], bufs.at[b], in_sems.at[b])

        def add_start(ch):
            b = ch % _SNBUF
            pltpu.async_copy(
                bufs.at[b], acc.at[idx_v.at[ch]], add_sems.at[b], add=True)

        def add_wait(ch):
            b = ch % _SNBUF
            pltpu.make_async_copy(
                bufs.at[b], acc.at[idx_v.at[ch]], add_sems.at[b]).wait()

        def body(ch, c):
            @pl.when(ch >= _SNBUF)
            def _():
                add_wait(ch - _SNBUF)
            in_desc(ch).start()

            @pl.when(ch >= _SQ)
            def _():
                in_desc(ch - _SQ).wait()
                add_start(ch - _SQ)
            return c

        lax.fori_loop(0, _CH, body, 0)

        def tail1(i, c):
            ch = _CH - _SQ + i
            in_desc(ch).wait()
            add_start(ch)
            return c

        lax.fori_loop(0, _SQ, tail1, 0)

        def tail2(i, c):
            add_wait(_CH - _SNBUF + i)
            return c

        lax.fori_loop(0, _SNBUF, tail2, 0)
        plsc.subcore_barrier()

        def drain_desc(k):
            r = sid * _RPT + k * _K
            return pltpu.make_async_copy(
                acc.at[pl.ds(r, _K)], out_hbm.at[cid, pl.ds(r, _K)],
                in_sems.at[k % _SNBUF])

        def drain(k, c):
            @pl.when(k >= _SNBUF)
            def _():
                drain_desc(k - _SNBUF).wait()
            drain_desc(k).start()
            return c

        lax.fori_loop(0, _RPT // _K, drain, 0)

        def drain_tail(i, c):
            drain_desc(_RPT // _K - _SNBUF + i).wait()
            return c

        lax.fori_loop(0, _SNBUF, drain_tail, 0)

    return gather_f32, scatter


def _mm_tc(xx, ww):
    nb = 10

    def kk(x_ref, w_ref, o_ref):
        o_ref[...] = jnp.dot(x_ref[...], w_ref[...],
                             preferred_element_type=F32)

    return pl.pallas_call(
        kk,
        compiler_params=_CP,
        grid=(nb,),
        in_specs=[pl.BlockSpec((_NP // nb, _D), lambda i: (i, 0)),
                  pl.BlockSpec((_D, _D), lambda i: (0, 0))],
        out_specs=pl.BlockSpec((_NP // nb, _D), lambda i: (i, 0)),
        out_shape=jax.ShapeDtypeStruct((_NP, _D), F32),
    )(xx, ww)


def _combine_tc(parts):
    nb = 10

    def kk(p_ref, o_ref):
        o_ref[...] = p_ref[0] + p_ref[1]

    return pl.pallas_call(
        kk,
        compiler_params=_CP,
        grid=(nb,),
        in_specs=[pl.BlockSpec((2, _NP // nb, _D), lambda i: (0, i, 0))],
        out_specs=pl.BlockSpec((_NP // nb, _D), lambda i: (i, 0)),
        out_shape=jax.ShapeDtypeStruct((_NP, _D), F32),
    )(parts)


def _init_tc(g0, e, wib):
    def kk(g_ref, e_ref, w_ref, inp_ref, m_ref):
        v = g_ref[...] + jnp.dot(e_ref[...], w_ref[...],
                                 preferred_element_type=F32)
        inp_ref[...] = v
        m_ref[...] = jnp.maximum(v, 0.0)

    return pl.pallas_call(
        kk,
        compiler_params=_CP,
        grid=(_NBLK,),
        in_specs=[pl.BlockSpec((_BLK, _D), lambda i: (i, 0)),
                  pl.BlockSpec((_BLK, 16), lambda i: (i, 0)),
                  pl.BlockSpec((16, _D), lambda i: (0, 0))],
        out_specs=[pl.BlockSpec((_BLK, _D), lambda i: (i, 0)),
                   pl.BlockSpec((_BLK, _D), lambda i: (i, 0))],
        out_shape=[jax.ShapeDtypeStruct((_E, _D), F32),
                   jax.ShapeDtypeStruct((_E, _D), F32)],
    )(g0, e, wib)


def _update_tc(inp, g, m, wh):
    def kk(inp_ref, g_ref, m_ref, w_ref, o_ref):
        gg = g_ref[...]
        mm = m_ref[...]
        up = jnp.roll(mm, -1, axis=0)
        dn = jnp.roll(mm, 1, axis=0)
        ridx = lax.broadcasted_iota(jnp.int32, (_BLK, _D), 0)
        msw = jnp.where(ridx % 2 == 0, up, dn)
        a = jnp.dot(gg - msw, w_ref[...], preferred_element_type=F32)
        o_ref[...] = jnp.maximum(inp_ref[...] + a, 0.0)

    return pl.pallas_call(
        kk,
        compiler_params=_CP,
        grid=(_NBLK,),
        in_specs=[pl.BlockSpec((_BLK, _D), lambda i: (i, 0)),
                  pl.BlockSpec((_BLK, _D), lambda i: (i, 0)),
                  pl.BlockSpec((_BLK, _D), lambda i: (i, 0)),
                  pl.BlockSpec((_D, _D), lambda i: (0, 0))],
        out_specs=pl.BlockSpec((_BLK, _D), lambda i: (i, 0)),
        out_shape=jax.ShapeDtypeStruct((_E, _D), F32),
    )(inp, g, m, wh)


def _final_tc(x, parts, seg8, wo1, wo2, b8, wmlp):
    def kk(x_ref, p_ref, s_ref, wo1_ref, wo2_ref, b_ref, wm_ref, o_ref):
        es = p_ref[0, :_N, :] + p_ref[1, :_N, :]
        h = (jnp.dot(x_ref[...], wo1_ref[...], preferred_element_type=F32)
             + jnp.dot(es, wo2_ref[...], preferred_element_type=F32)
             + b_ref[0:1, :])
        h = jnp.maximum(h, 0.0)
        diff = h[_NHALF:, :] - h[:_NHALF, :]
        t = jnp.maximum(jnp.dot(diff, wm_ref[...],
                                preferred_element_type=F32), 0.0)
        seg = jnp.broadcast_to(s_ref[0:1, :], (128, _NHALF))
        oh = (seg == lax.broadcasted_iota(jnp.int32, (128, _NHALF), 0))
        o_ref[...] = jnp.dot(oh.astype(F32), t, preferred_element_type=F32)

    return pl.pallas_call(
        kk,
        compiler_params=_CP,
        out_shape=jax.ShapeDtypeStruct((128, _D), F32),
    )(x, parts, seg8, wo1, wo2, b8, wmlp)


def kernel(x, e, edge_index, segment_ids, W_i, W_h, W_o, b_o, W_mlp):
    src = edge_index[0]
    dst = edge_index[1]
    src3 = src.reshape(_NW, _CH, _K)
    dst3 = dst.reshape(_NW, _CH, _K)
    seg8 = jnp.tile(segment_ids[None, :], (8, 1))
    b8 = jnp.tile(b_o[None, :], (8, 1))

    gather_f32, scatter = _sc_kernels()

    xp = jnp.concatenate([x, jnp.zeros((_NP - _N, _ATOM), F32)], axis=0)
    a = _mm_tc(xp, W_i[:_ATOM])
    g0 = gather_f32(a, src3)
    inp, m = _init_tc(g0, e, W_i[_ATOM:])
    for _ in range(4):
        parts = scatter(m, dst3)
        es = _combine_tc(parts)
        g = gather_f32(es, src3)
        m = _update_tc(inp, g, m, W_h)
    parts = scatter(m, dst3)
    out = _final_tc(x, parts, seg8, W_o[:_ATOM], W_o[_ATOM:], b8, W_mlp)
    return out[:100]


# speedup vs baseline: 1.1178x; 1.0028x over previous
import functools

import jax
import jax.numpy as jnp
from jax import lax
from jax.experimental import pallas as pl
from jax.experimental.pallas import tpu as pltpu
from jax.experimental.pallas import tpu_sc as plsc

F32 = jnp.float32
_CP = pltpu.CompilerParams(skip_device_barrier=True)

_N = 10000
_E = 320000
_D = 128
_ATOM = 128
_NHALF = _N // 2

_NC = 2
_NS = 16
_NW = _NC * _NS
_NP = 10240
_RPT = _NP // _NS

_K = 80
_PER_W = _E // _NW
_CH = _PER_W // _K

_GNBUF = 10
_GQ = 5
_SNBUF = 3
_SQ = 2

_BLK = 3200
_NBLK = _E // _BLK


@functools.cache
def _sc_kernels():
    mesh = plsc.VectorSubcoreMesh(
        core_axis_name="c", subcore_axis_name="s", num_cores=_NC,
        num_subcores=_NS)

    def make_gather(ncols, dtype, nbuf, q):
        @functools.partial(
            pl.kernel,
            out_type=jax.ShapeDtypeStruct((_E, ncols), dtype),
            mesh=mesh,
            compiler_params=_CP,
            scratch_types=[
                pltpu.VMEM((_CH, _K), jnp.int32),
                pltpu.VMEM((nbuf, _K, ncols), dtype),
                pltpu.SemaphoreType.DMA((nbuf,)),
                pltpu.SemaphoreType.DMA((nbuf,)),
            ],
        )
        def gather(tab_hbm, idx_hbm, out_hbm, idx_v, bufs, in_sems, out_sems):
            cid = lax.axis_index("c")
            sid = lax.axis_index("s")
            wid = sid * _NC + cid
            pltpu.sync_copy(idx_hbm.at[wid], idx_v)

            def in_desc(ch):
                b = ch % nbuf
                return pltpu.make_async_copy(
                    tab_hbm.at[idx_v.at[ch]], bufs.at[b], in_sems.at[b])

            def out_desc(ch):
                b = ch % nbuf
                base = wid * _PER_W + ch * _K
                return pltpu.make_async_copy(
                    bufs.at[b], out_hbm.at[pl.ds(base, _K)], out_sems.at[b])

            def body(ch, c):
                @pl.when(ch >= nbuf)
                def _():
                    out_desc(ch - nbuf).wait()
                in_desc(ch).start()

                @pl.when(ch >= q)
                def _():
                    in_desc(ch - q).wait()
                    out_desc(ch - q).start()
                return c

            lax.fori_loop(0, _CH, body, 0)

            def tail1(i, c):
                ch = _CH - q + i
                in_desc(ch).wait()
                out_desc(ch).start()
                return c

            lax.fori_loop(0, q, tail1, 0)

            def tail2(i, c):
                out_desc(_CH - nbuf + i).wait()
                return c

            lax.fori_loop(0, nbuf, tail2, 0)

        return gather

    gather_f32 = make_gather(_D, F32, _GNBUF, _GQ)

    @functools.partial(
        pl.kernel,
        out_type=jax.ShapeDtypeStruct((_NC, _NP, _D), F32),
        mesh=mesh,
        compiler_params=_CP,
        scratch_types=[
            pltpu.VMEM((_CH, _K), jnp.int32),
            pltpu.VMEM((_SNBUF, _K, _D), F32),
            pltpu.VMEM_SHARED((_NP, _D), F32),
            pltpu.SemaphoreType.DMA((_SNBUF,)),
            pltpu.SemaphoreType.DMA((_SNBUF,)),
        ],
    )
    def scatter(msg_hbm, dst_hbm, out_hbm, idx_v, bufs, acc,
                in_sems, add_sems):
        cid = lax.axis_index("c")
        sid = lax.axis_index("s")
        wid = sid * _NC + cid
        z16 = jnp.zeros((16,), F32)

        def zrow(i, c):
            for j in range(8):
                bufs[0, i, pl.ds(j * 16, 16)] = z16
            return c

        lax.fori_loop(0, _K, zrow, 0)

        def zdesc(k):
            return pltpu.make_async_copy(
                bufs.at[0], acc.at[pl.ds(sid * _RPT + k * _K, _K)],
                add_sems.at[k % _SNBUF])

        def zacc(k, c):
            @pl.when(k >= _SNBUF)
            def _():
                zdesc(k - _SNBUF).wait()
            zdesc(k).start()
            return c

        lax.fori_loop(0, _RPT // _K, zacc, 0)
        pltpu.sync_copy(dst_hbm.at[wid], idx_v)

        def ztail(i, c):
            zdesc(_RPT // _K - _SNBUF + i).wait()
            return c

        lax.fori_loop(0, _SNBUF, ztail, 0)
        plsc.subcore_barrier()

        def in_desc(ch):
            b = ch % _SNBUF
            base = wid * _PER_W + ch * _K
            return pltpu.make_async_copy(
                msg_hbm.at[pl.ds(base, _K)], bufs.at[b], in_sems.at[b])

        def add_start(ch):
            b = ch % _SNBUF
            pltpu.async_copy(
                bufs.at[b], acc.at[idx_v.at[ch]], add_sems.at[b], add=True)

        def add_wait(ch):
            b = ch % _SNBUF
            pltpu.make_async_copy(
                bufs.at[b], acc.at[idx_v.at[ch]], add_sems.at[b]).wait()

        def body(ch, c):
            @pl.when(ch >= _SNBUF)
            def _():
                add_wait(ch - _SNBUF)
            in_desc(ch).start()

            @pl.when(ch >= _SQ)
            def _():
                in_desc(ch - _SQ).wait()
                add_start(ch - _SQ)
            return c

        lax.fori_loop(0, _CH, body, 0)

        def tail1(i, c):
            ch = _CH - _SQ + i
            in_desc(ch).wait()
            add_start(ch)
            return c

        lax.fori_loop(0, _SQ, tail1, 0)

        def tail2(i, c):
            add_wait(_CH - _SNBUF + i)
            return c

        lax.fori_loop(0, _SNBUF, tail2, 0)
        plsc.subcore_barrier()

        def drain_desc(k):
            r = sid * _RPT + k * _K
            return pltpu.make_async_copy(
                acc.at[pl.ds(r, _K)], out_hbm.at[cid, pl.ds(r, _K)],
                in_sems.at[k % _SNBUF])

        def drain(k, c):
            @pl.when(k >= _SNBUF)
            def _():
                drain_desc(k - _SNBUF).wait()
            drain_desc(k).start()
            return c

        lax.fori_loop(0, _RPT // _K, drain, 0)

        def drain_tail(i, c):
            drain_desc(_RPT // _K - _SNBUF + i).wait()
            return c

        lax.fori_loop(0, _SNBUF, drain_tail, 0)

    return gather_f32, scatter


def _mm_tc(xx, ww):
    nb = 10

    def kk(x_ref, w_ref, o_ref):
        o_ref[...] = jnp.dot(x_ref[...], w_ref[...],
                             preferred_element_type=F32)

    return pl.pallas_call(
        kk,
        compiler_params=_CP,
        grid=(nb,),
        in_specs=[pl.BlockSpec((_NP // nb, _D), lambda i: (i, 0)),
                  pl.BlockSpec((_D, _D), lambda i: (0, 0))],
        out_specs=pl.BlockSpec((_NP // nb, _D), lambda i: (i, 0)),
        out_shape=jax.ShapeDtypeStruct((_NP, _D), F32),
    )(xx, ww)


def _combine_tc(parts):
    nb = 10

    def kk(p_ref, o_ref):
        o_ref[...] = p_ref[0] + p_ref[1]

    return pl.pallas_call(
        kk,
        compiler_params=_CP,
        grid=(nb,),
        in_specs=[pl.BlockSpec((2, _NP // nb, _D), lambda i: (0, i, 0))],
        out_specs=pl.BlockSpec((_NP // nb, _D), lambda i: (i, 0)),
        out_shape=jax.ShapeDtypeStruct((_NP, _D), F32),
    )(parts)


def _init_tc(g0, e, wib):
    def kk(g_ref, e_ref, w_ref, inp_ref, m_ref):
        v = g_ref[...] + jnp.dot(e_ref[...], w_ref[...],
                                 preferred_element_type=F32)
        inp_ref[...] = v
        m_ref[...] = jnp.maximum(v, 0.0)

    return pl.pallas_call(
        kk,
        compiler_params=_CP,
        grid=(_NBLK,),
        in_specs=[pl.BlockSpec((_BLK, _D), lambda i: (i, 0)),
                  pl.BlockSpec((_BLK, 16), lambda i: (i, 0)),
                  pl.BlockSpec((16, _D), lambda i: (0, 0))],
        out_specs=[pl.BlockSpec((_BLK, _D), lambda i: (i, 0)),
                   pl.BlockSpec((_BLK, _D), lambda i: (i, 0))],
        out_shape=[jax.ShapeDtypeStruct((_E, _D), F32),
                   jax.ShapeDtypeStruct((_E, _D), F32)],
    )(g0, e, wib)


def _update_tc(inp, g, m, wh):
    def kk(inp_ref, g_ref, m_ref, w_ref, o_ref):
        gg = g_ref[...]
        mm = m_ref[...]
        up = jnp.roll(mm, -1, axis=0)
        dn = jnp.roll(mm, 1, axis=0)
        ridx = lax.broadcasted_iota(jnp.int32, (_BLK, _D), 0)
        msw = jnp.where(ridx % 2 == 0, up, dn)
        a = jnp.dot(gg - msw, w_ref[...], preferred_element_type=F32)
        o_ref[...] = jnp.maximum(inp_ref[...] + a, 0.0)

    return pl.pallas_call(
        kk,
        compiler_params=_CP,
        grid=(_NBLK,),
        in_specs=[pl.BlockSpec((_BLK, _D), lambda i: (i, 0)),
                  pl.BlockSpec((_BLK, _D), lambda i: (i, 0)),
                  pl.BlockSpec((_BLK, _D), lambda i: (i, 0)),
                  pl.BlockSpec((_D, _D), lambda i: (0, 0))],
        out_specs=pl.BlockSpec((_BLK, _D), lambda i: (i, 0)),
        out_shape=jax.ShapeDtypeStruct((_E, _D), F32),
    )(inp, g, m, wh)


def _final_tc(x, parts, seg8, wo1, wo2, b8, wmlp):
    def kk(x_ref, p_ref, s_ref, wo1_ref, wo2_ref, b_ref, wm_ref, o_ref):
        es = p_ref[0, :_N, :] + p_ref[1, :_N, :]
        h = (jnp.dot(x_ref[...], wo1_ref[...], preferred_element_type=F32)
             + jnp.dot(es, wo2_ref[...], preferred_element_type=F32)
             + b_ref[0:1, :])
        h = jnp.maximum(h, 0.0)
        diff = h[_NHALF:, :] - h[:_NHALF, :]
        t = jnp.maximum(jnp.dot(diff, wm_ref[...],
                                preferred_element_type=F32), 0.0)
        seg = jnp.broadcast_to(s_ref[0:1, :], (128, _NHALF))
        oh = (seg == lax.broadcasted_iota(jnp.int32, (128, _NHALF), 0))
        o_ref[...] = jnp.dot(oh.astype(F32), t, preferred_element_type=F32)

    return pl.pallas_call(
        kk,
        compiler_params=_CP,
        out_shape=jax.ShapeDtypeStruct((128, _D), F32),
    )(x, parts, seg8, wo1, wo2, b8, wmlp)


def kernel(x, e, edge_index, segment_ids, W_i, W_h, W_o, b_o, W_mlp):
    src = edge_index[0]
    dst = edge_index[1]
    src3 = src.reshape(_NW, _CH, _K)
    dst3 = dst.reshape(_NW, _CH, _K)
    seg8 = jnp.tile(segment_ids[None, :], (8, 1))
    b8 = jnp.tile(b_o[None, :], (8, 1))

    gather_f32, scatter = _sc_kernels()

    xp = jnp.concatenate([x, jnp.zeros((_NP - _N, _ATOM), F32)], axis=0)
    a = _mm_tc(xp, W_i[:_ATOM])
    g0 = gather_f32(a, src3)
    inp, m = _init_tc(g0, e, W_i[_ATOM:])
    for _ in range(4):
        parts = scatter(m, dst3)
        es = _combine_tc(parts)
        g = gather_f32(es, src3)
        m = _update_tc(inp, g, m, W_h)
    parts = scatter(m, dst3)
    out = _final_tc(x, parts, seg8, W_o[:_ATOM], W_o[_ATOM:], b8, W_mlp)
    return out[:100]
